# baseline (speedup 1.0000x reference)
import functools

import jax
import jax.numpy as jnp
import numpy as np
from jax import lax
from jax.experimental import pallas as pl
from jax.experimental.pallas import tpu as pltpu
from jax.experimental.pallas import tpu_sc as plsc

N = 10000
E = 320000
H = 128
RBF_DIM = 20
NLAYER = 3
CUTOFF = 5.0
C = 224
V = 480
G = H + 2 * C

NC = 2
NS = 16
NW = NC * NS
LANES = 16

EB = 40
EPT = E // NW
NBLK = EPT // EB
NBLKP = 256

GR = EPT // LANES
GRP = 640

WC = 128
_VCHUNKS = [(0, 128), (128, 256), (256, 384), (384, 480)]

_cc = np.concatenate([
    np.arange(128),
    np.repeat(128 + np.arange(64), 3),
    np.repeat(192 + np.arange(32), 5),
]).astype(np.int32)
_compv = np.concatenate([
    np.zeros(128, np.int32),
    1 + np.tile(np.arange(3, dtype=np.int32), 64),
    4 + np.tile(np.arange(5, dtype=np.int32), 32),
])


def _mesh():
    return plsc.VectorSubcoreMesh(
        core_axis_name="c", subcore_axis_name="s",
        num_cores=NC, num_subcores=NS)


def _worker_id():
    return lax.axis_index("c") * NS + lax.axis_index("s")


def _geom_body(posx_hbm, posy_hbm, posz_hbm, srcg_hbm, dstg_hbm,
               bx_hbm, by_hbm, bz_hbm,
               tabx, taby, tabz, src8, dst8, ox8, oy8, oz8):
    wid = _worker_id()
    pltpu.sync_copy(posx_hbm, tabx)
    pltpu.sync_copy(posy_hbm, taby)
    pltpu.sync_copy(posz_hbm, tabz)

    def group(g, carry):
        pltpu.sync_copy(srcg_hbm.at[wid, pl.ds(g * 8, 8)], src8)
        pltpu.sync_copy(dstg_hbm.at[wid, pl.ds(g * 8, 8)], dst8)
        for k in range(8):
            isrc = src8[k]
            idst = dst8[k]
            ox8[k] = (plsc.load_gather(tabx, [isrc])
                      - plsc.load_gather(tabx, [idst]))
            oy8[k] = (plsc.load_gather(taby, [isrc])
                      - plsc.load_gather(taby, [idst]))
            oz8[k] = (plsc.load_gather(tabz, [isrc])
                      - plsc.load_gather(tabz, [idst]))
        pltpu.sync_copy(ox8, bx_hbm.at[wid, pl.ds(g * 8, 8)])
        pltpu.sync_copy(oy8, by_hbm.at[wid, pl.ds(g * 8, 8)])
        pltpu.sync_copy(oz8, bz_hbm.at[wid, pl.ds(g * 8, 8)])
        return carry

    lax.fori_loop(0, GRP // 8, group, 0)


@functools.lru_cache(maxsize=None)
def _build_geom_kernel():
    o3 = jax.ShapeDtypeStruct((NW, GRP, LANES), jnp.float32)
    return functools.partial(
        pl.kernel,
        out_type=(o3, o3, o3),
        mesh=_mesh(),
        compiler_params=pltpu.CompilerParams(needs_layout_passes=False),
        scratch_types=[
            pltpu.VMEM((N,), jnp.float32),
            pltpu.VMEM((N,), jnp.float32),
            pltpu.VMEM((N,), jnp.float32),
            pltpu.VMEM((8, LANES), jnp.int32),
            pltpu.VMEM((8, LANES), jnp.int32),
            pltpu.VMEM((8, LANES), jnp.float32),
            pltpu.VMEM((8, LANES), jnp.float32),
            pltpu.VMEM((8, LANES), jnp.float32),
        ],
    )(_geom_body)


def _geom_call(posx, posy, posz, srcg, dstg):
    return _build_geom_kernel()(posx, posy, posz, srcg, dstg)


def _make_seg_kernel(has_b):
    nj = WC // LANES

    def body(*refs):
        if has_b:
            (t1_hbm, t2_hbm, a_hbm, b_hbm, zero_hbm, srcm_hbm, dstm_hbm,
             out_hbm, srcx, dstx, a_v, b_v, r1_v, r2_v, accum,
             sem0, sem1) = refs
        else:
            (t1_hbm, a_hbm, zero_hbm, srcm_hbm, dstm_hbm,
             out_hbm, srcx, dstx, a_v, r1_v, accum, sem0, sem1) = refs
            b_v = r2_v = None
        sems = (sem0, sem1)
        cid = lax.axis_index("c")
        sid = lax.axis_index("s")
        wid = cid * NS + sid

        @pl.when(sid == 0)
        def _init():
            pltpu.sync_copy(zero_hbm, accum)

        plsc.subcore_barrier()

        def issue(b, gslot, k, ph):
            base = wid * EPT + b * EB
            idx = srcx.at[gslot, k]
            pltpu.async_copy(t1_hbm.at[idx], r1_v.at[ph], sems[ph])
            pltpu.async_copy(a_hbm.at[pl.ds(base, EB)], a_v.at[ph],
                             sems[ph])
            if has_b:
                pltpu.async_copy(t2_hbm.at[idx], r2_v.at[ph], sems[ph])
                pltpu.async_copy(b_hbm.at[pl.ds(base, EB)], b_v.at[ph],
                                 sems[ph])

        def drain(b, gslot, k, ph):
            base = wid * EPT + b * EB
            idx = srcx.at[gslot, k]
            pltpu.make_async_copy(t1_hbm.at[idx], r1_v.at[ph],
                                  sems[ph]).wait()
            pltpu.make_async_copy(a_hbm.at[pl.ds(base, EB)], a_v.at[ph],
                                  sems[ph]).wait()
            if has_b:
                pltpu.make_async_copy(t2_hbm.at[idx], r2_v.at[ph],
                                      sems[ph]).wait()
                pltpu.make_async_copy(b_hbm.at[pl.ds(base, EB)],
                                      b_v.at[ph], sems[ph]).wait()

        def compute(ph):
            def row(i, carry2):
                for j in range(nj):
                    sl = pl.ds(j * LANES, LANES)
                    m = a_v[ph, i, sl] * r1_v[ph, i, sl]
                    if has_b:
                        m = m + b_v[ph, i, sl] * r2_v[ph, i, sl]
                    r1_v[ph, i, sl] = m
                return carry2

            lax.fori_loop(0, EB, row, 0)

        pltpu.sync_copy(srcm_hbm.at[wid, pl.ds(0, 8)], srcx.at[0])
        pltpu.sync_copy(dstm_hbm.at[wid, pl.ds(0, 8)], dstx.at[0])
        issue(0, 0, 0, 0)

        def group(g, carry):
            gcur = g & 1
            gnext = 1 - gcur

            @pl.when((g + 1) * 8 < NBLKP)
            def _prefetch_idx():
                pltpu.sync_copy(srcm_hbm.at[wid, pl.ds((g + 1) * 8, 8)],
                                srcx.at[gnext])
                pltpu.sync_copy(dstm_hbm.at[wid, pl.ds((g + 1) * 8, 8)],
                                dstx.at[gnext])
            for k in range(8):
                b = g * 8 + k
                ph = k & 1

                @pl.when(b + 1 < NBLK)
                def _issue_next(b=b, k=k, ph=ph, gcur=gcur, gnext=gnext):
                    if k + 1 < 8:
                        issue(b + 1, gcur, k + 1, 1 - ph)
                    else:
                        issue(b + 1, gnext, 0, 1 - ph)

                @pl.when(b < NBLK)
                def _process(b=b, k=k, ph=ph, gcur=gcur):
                    drain(b, gcur, k, ph)
                    compute(ph)
                    pltpu.sync_copy(r1_v.at[ph],
                                    accum.at[dstx.at[gcur, k]], add=True)
            return carry

        lax.fori_loop(0, NBLKP // 8, group, 0)
        plsc.subcore_barrier()

        @pl.when(sid == 0)
        def _writeout():
            pltpu.sync_copy(accum, out_hbm.at[cid])

    scratch = [
        pltpu.VMEM((2, 8, EB), jnp.int32),
        pltpu.VMEM((2, 8, EB), jnp.int32),
        pltpu.VMEM((2, EB, WC), jnp.float32),
    ]
    if has_b:
        scratch.append(pltpu.VMEM((2, EB, WC), jnp.float32))
    scratch.append(pltpu.VMEM((2, EB, WC), jnp.float32))
    if has_b:
        scratch.append(pltpu.VMEM((2, EB, WC), jnp.float32))
    scratch += [
        pltpu.VMEM_SHARED((N, WC), jnp.float32),
        pltpu.SemaphoreType.DMA,
        pltpu.SemaphoreType.DMA,
    ]
    return functools.partial(
        pl.kernel,
        out_type=jax.ShapeDtypeStruct((NC, N, WC), jnp.float32),
        mesh=_mesh(),
        scratch_types=scratch,
    )(body)


@functools.lru_cache(maxsize=None)
def _seg_kernel_cached(has_b):
    return _make_seg_kernel(has_b)


def _seg_call(has_b, *args):
    return _seg_kernel_cached(has_b)(*args)


def _silu(x):
    return x * jax.nn.sigmoid(x)


def kernel(atom_pos, x, Wp1, bp1, Wp2, bp2, Wf, bf, Wu1, bu1, Wu2, bu2,
           edge_index):
    cc = jnp.asarray(_cc)

    src = edge_index[0].astype(jnp.int32)
    dst = edge_index[1].astype(jnp.int32)
    srcm = jnp.pad(src.reshape(NW, NBLK, EB),
                   ((0, 0), (0, NBLKP - NBLK), (0, 0)))
    dstm = jnp.pad(dst.reshape(NW, NBLK, EB),
                   ((0, 0), (0, NBLKP - NBLK), (0, 0)))
    srcg = jnp.pad(src.reshape(NW, GR, LANES),
                   ((0, 0), (0, GRP - GR), (0, 0)))
    dstg = jnp.pad(dst.reshape(NW, GR, LANES),
                   ((0, 0), (0, GRP - GR), (0, 0)))

    bxm, bym, bzm = _geom_call(
        atom_pos[:, 0], atom_pos[:, 1], atom_pos[:, 2], srcg, dstg)
    bx = bxm[:, :GR].reshape(E)
    by = bym[:, :GR].reshape(E)
    bz = bzm[:, :GR].reshape(E)

    d2 = bx * bx + by * by + bz * bz + 1e-12
    d = jnp.sqrt(d2)
    u = d / CUTOFF
    env = jnp.where(
        u < 1.0, 1.0 - 21.0 * u**5 + 35.0 * u**6 - 15.0 * u**7, 0.0)
    nvec = jnp.arange(1, RBF_DIM + 1, dtype=jnp.float32)
    rbf = (jnp.sqrt(2.0 / CUTOFF)
           * jnp.sin(nvec[None, :] * jnp.pi * u[:, None]) / d[:, None])
    fE = rbf * env[:, None]
    ux, uy, uz = bx / d, by / d, bz / d
    s3 = jnp.sqrt(3.0)
    Y9 = jnp.stack([
        jnp.ones_like(ux), ux, uy, uz,
        s3 * ux * uy, s3 * uy * uz, 0.5 * (3.0 * uz * uz - 1.0),
        s3 * ux * uz, 0.5 * s3 * (ux * ux - uy * uy),
    ], axis=1)

    zeros_c = jnp.zeros((N, WC), jnp.float32)

    def padc(arr):
        w = arr.shape[1]
        if w == WC:
            return arr
        return jnp.pad(arr, ((0, 0), (0, WC - w)))

    AS, AK, BK = [], [], []
    for l in range(NLAYER):
        AS.append(fE @ Wf[l][:, :H] + env[:, None] * bf[l][None, :H])
        aks, bks = [], []
        for (j0, j1) in _VCHUNKS:
            cck = _cc[j0:j1]
            compk = _compv[j0:j1]
            wf2 = Wf[l][:, H + cck]
            bf2 = bf[l][H + cck]
            yk = Y9[:, compk]
            aks.append(padc((fE @ wf2 + env[:, None] * bf2[None, :]) * yk))
            if l > 0:
                wf3 = Wf[l][:, H + C + cck]
                bf3 = bf[l][H + C + cck]
                bks.append(padc(fE @ wf3 + env[:, None] * bf3[None, :]))
        AK.append(aks)
        BK.append(bks)

    s = x
    v = jnp.zeros((N, V), jnp.float32)
    for l in range(NLAYER):
        phi = _silu(s @ Wp1[l] + bp1[l]) @ Wp2[l] + bp2[l]

        out = _seg_call(False, phi[:, :H], AS[l], zeros_c, srcm, dstm)
        ds = out[0] + out[1]
        s = s + ds

        dv_parts = []
        for ci, (j0, j1) in enumerate(_VCHUNKS):
            w = j1 - j0
            cck = _cc[j0:j1]
            t1k = padc(phi[:, H + cck])
            if l == 0:
                out = _seg_call(False, t1k, AK[l][ci], zeros_c, srcm, dstm)
            else:
                t2k = padc(phi[:, H + C + cck] * v[:, j0:j1])
                out = _seg_call(True, t1k, t2k, AK[l][ci], BK[l][ci],
                                zeros_c, srcm, dstm)
            dv_parts.append((out[0] + out[1])[:, :w])
        v = v + jnp.concatenate(dv_parts, axis=1)

        vn = jnp.concatenate([
            jnp.sqrt(v[:, :128] ** 2 + 1e-12),
            jnp.sqrt(jnp.sum(v[:, 128:320].reshape(N, 64, 3) ** 2, -1)
                     + 1e-12),
            jnp.sqrt(jnp.sum(v[:, 320:480].reshape(N, 32, 5) ** 2, -1)
                     + 1e-12),
        ], axis=1)
        a = (_silu(jnp.concatenate([s, vn], axis=1) @ Wu1[l] + bu1[l])
             @ Wu2[l] + bu2[l])
        s = s + a[:, :H]
        v = v + a[:, H:][:, cc] * v
    return s


# speedup vs baseline: 1.0386x; 1.0386x over previous
import functools

import jax
import jax.numpy as jnp
import numpy as np
from jax import lax
from jax.experimental import pallas as pl
from jax.experimental.pallas import tpu as pltpu
from jax.experimental.pallas import tpu_sc as plsc

N = 10000
E = 320000
H = 128
RBF_DIM = 20
NLAYER = 3
CUTOFF = 5.0
C = 224
V = 480
G = H + 2 * C

NC = 2
NS = 16
NW = NC * NS
LANES = 16

EB = 40
EPT = E // NW
NBLK = EPT // EB
NBLKP = 256

GR = EPT // LANES
GRP = 640

WC = 128
_VCHUNKS = [(0, 128), (128, 256), (256, 384), (384, 480)]

_cc = np.concatenate([
    np.arange(128),
    np.repeat(128 + np.arange(64), 3),
    np.repeat(192 + np.arange(32), 5),
]).astype(np.int32)
_compv = np.concatenate([
    np.zeros(128, np.int32),
    1 + np.tile(np.arange(3, dtype=np.int32), 64),
    4 + np.tile(np.arange(5, dtype=np.int32), 32),
])


def _mesh():
    return plsc.VectorSubcoreMesh(
        core_axis_name="c", subcore_axis_name="s",
        num_cores=NC, num_subcores=NS)


def _worker_id():
    return lax.axis_index("c") * NS + lax.axis_index("s")


def _geom_body(posx_hbm, posy_hbm, posz_hbm, srcg_hbm, dstg_hbm,
               bx_hbm, by_hbm, bz_hbm,
               tabx, taby, tabz, src8, dst8, ox8, oy8, oz8):
    wid = _worker_id()
    pltpu.sync_copy(posx_hbm, tabx)
    pltpu.sync_copy(posy_hbm, taby)
    pltpu.sync_copy(posz_hbm, tabz)

    def group(g, carry):
        pltpu.sync_copy(srcg_hbm.at[wid, pl.ds(g * 8, 8)], src8)
        pltpu.sync_copy(dstg_hbm.at[wid, pl.ds(g * 8, 8)], dst8)
        for k in range(8):
            isrc = src8[k]
            idst = dst8[k]
            ox8[k] = (plsc.load_gather(tabx, [isrc])
                      - plsc.load_gather(tabx, [idst]))
            oy8[k] = (plsc.load_gather(taby, [isrc])
                      - plsc.load_gather(taby, [idst]))
            oz8[k] = (plsc.load_gather(tabz, [isrc])
                      - plsc.load_gather(tabz, [idst]))
        pltpu.sync_copy(ox8, bx_hbm.at[wid, pl.ds(g * 8, 8)])
        pltpu.sync_copy(oy8, by_hbm.at[wid, pl.ds(g * 8, 8)])
        pltpu.sync_copy(oz8, bz_hbm.at[wid, pl.ds(g * 8, 8)])
        return carry

    lax.fori_loop(0, GRP // 8, group, 0)


@functools.lru_cache(maxsize=None)
def _build_geom_kernel():
    o3 = jax.ShapeDtypeStruct((NW, GRP, LANES), jnp.float32)
    return functools.partial(
        pl.kernel,
        out_type=(o3, o3, o3),
        mesh=_mesh(),
        compiler_params=pltpu.CompilerParams(needs_layout_passes=False),
        scratch_types=[
            pltpu.VMEM((N,), jnp.float32),
            pltpu.VMEM((N,), jnp.float32),
            pltpu.VMEM((N,), jnp.float32),
            pltpu.VMEM((8, LANES), jnp.int32),
            pltpu.VMEM((8, LANES), jnp.int32),
            pltpu.VMEM((8, LANES), jnp.float32),
            pltpu.VMEM((8, LANES), jnp.float32),
            pltpu.VMEM((8, LANES), jnp.float32),
        ],
    )(_geom_body)


def _geom_call(posx, posy, posz, srcg, dstg):
    return _build_geom_kernel()(posx, posy, posz, srcg, dstg)


def _make_seg_kernel(has_b):
    nj = WC // LANES

    def body(*refs):
        if has_b:
            (t1_hbm, a_hbm, zero_hbm, srcm_hbm, dstm_hbm,
             out_hbm, srcx, dstx, a_v, r1_v, msg_v, accum,
             sem0, sem1) = refs
        else:
            (t1_hbm, a_hbm, zero_hbm, srcm_hbm, dstm_hbm,
             out_hbm, srcx, dstx, a_v, r1_v, accum, sem0, sem1) = refs
            msg_v = None
        sems = (sem0, sem1)
        cid = lax.axis_index("c")
        sid = lax.axis_index("s")
        wid = cid * NS + sid

        @pl.when(sid == 0)
        def _init():
            pltpu.sync_copy(zero_hbm, accum)

        plsc.subcore_barrier()

        def issue(b, gslot, k, ph):
            base = wid * EPT + b * EB
            idx = srcx.at[gslot, k]
            pltpu.async_copy(t1_hbm.at[idx], r1_v.at[ph], sems[ph])
            pltpu.async_copy(a_hbm.at[pl.ds(base, EB)], a_v.at[ph],
                             sems[ph])

        def drain(b, gslot, k, ph):
            base = wid * EPT + b * EB
            idx = srcx.at[gslot, k]
            pltpu.make_async_copy(t1_hbm.at[idx], r1_v.at[ph],
                                  sems[ph]).wait()
            pltpu.make_async_copy(a_hbm.at[pl.ds(base, EB)], a_v.at[ph],
                                  sems[ph]).wait()

        hi_mask = np.int32(np.uint32(0xFFFF0000).view(np.int32))

        def compute(ph):
            def row(i, carry2):
                for j in range(nj):
                    sl = pl.ds(j * LANES, LANES)
                    if has_b:
                        ab = a_v[ph, i, sl]
                        t12 = r1_v[ph, i, sl]
                        af = plsc.bitcast(ab << 16, jnp.float32)
                        bf16_hi = plsc.bitcast(ab & hi_mask, jnp.float32)
                        t1f = plsc.bitcast(t12 << 16, jnp.float32)
                        t2f = plsc.bitcast(t12 & hi_mask, jnp.float32)
                        msg_v[i, sl] = af * t1f + bf16_hi * t2f
                    else:
                        r1_v[ph, i, sl] = a_v[ph, i, sl] * r1_v[ph, i, sl]
                return carry2

            lax.fori_loop(0, EB, row, 0)

        pltpu.sync_copy(srcm_hbm.at[wid, pl.ds(0, 8)], srcx.at[0])
        pltpu.sync_copy(dstm_hbm.at[wid, pl.ds(0, 8)], dstx.at[0])
        issue(0, 0, 0, 0)

        def group(g, carry):
            gcur = g & 1
            gnext = 1 - gcur

            @pl.when((g + 1) * 8 < NBLKP)
            def _prefetch_idx():
                pltpu.sync_copy(srcm_hbm.at[wid, pl.ds((g + 1) * 8, 8)],
                                srcx.at[gnext])
                pltpu.sync_copy(dstm_hbm.at[wid, pl.ds((g + 1) * 8, 8)],
                                dstx.at[gnext])
            for k in range(8):
                b = g * 8 + k
                ph = k & 1

                @pl.when(b + 1 < NBLK)
                def _issue_next(b=b, k=k, ph=ph, gcur=gcur, gnext=gnext):
                    if k + 1 < 8:
                        issue(b + 1, gcur, k + 1, 1 - ph)
                    else:
                        issue(b + 1, gnext, 0, 1 - ph)

                @pl.when(b < NBLK)
                def _process(b=b, k=k, ph=ph, gcur=gcur):
                    drain(b, gcur, k, ph)
                    compute(ph)
                    msg = msg_v if has_b else r1_v.at[ph]
                    pltpu.sync_copy(msg,
                                    accum.at[dstx.at[gcur, k]], add=True)
            return carry

        lax.fori_loop(0, NBLKP // 8, group, 0)
        plsc.subcore_barrier()

        @pl.when(sid == 0)
        def _writeout():
            pltpu.sync_copy(accum, out_hbm.at[cid])

    dt = jnp.int32 if has_b else jnp.float32
    scratch = [
        pltpu.VMEM((2, 8, EB), jnp.int32),
        pltpu.VMEM((2, 8, EB), jnp.int32),
        pltpu.VMEM((2, EB, WC), dt),
        pltpu.VMEM((2, EB, WC), dt),
    ]
    if has_b:
        scratch.append(pltpu.VMEM((EB, WC), jnp.float32))
    scratch += [
        pltpu.VMEM_SHARED((N, WC), jnp.float32),
        pltpu.SemaphoreType.DMA,
        pltpu.SemaphoreType.DMA,
    ]
    return functools.partial(
        pl.kernel,
        out_type=jax.ShapeDtypeStruct((NC, N, WC), jnp.float32),
        mesh=_mesh(),
        compiler_params=pltpu.CompilerParams(needs_layout_passes=False),
        scratch_types=scratch,
    )(body)


@functools.lru_cache(maxsize=None)
def _seg_kernel_cached(has_b):
    return _make_seg_kernel(has_b)


def _seg_call(has_b, *args):
    return _seg_kernel_cached(has_b)(*args)


def _silu(x):
    return x * jax.nn.sigmoid(x)


def _pack2(a, b):
    au = jax.lax.bitcast_convert_type(
        a.astype(jnp.bfloat16), jnp.uint16).astype(jnp.uint32)
    bu = jax.lax.bitcast_convert_type(
        b.astype(jnp.bfloat16), jnp.uint16).astype(jnp.uint32)
    return jax.lax.bitcast_convert_type(au | (bu << 16), jnp.int32)


def kernel(atom_pos, x, Wp1, bp1, Wp2, bp2, Wf, bf, Wu1, bu1, Wu2, bu2,
           edge_index):
    cc = jnp.asarray(_cc)

    src = edge_index[0].astype(jnp.int32)
    dst = edge_index[1].astype(jnp.int32)
    srcm = jnp.pad(src.reshape(NW, NBLK, EB),
                   ((0, 0), (0, NBLKP - NBLK), (0, 0)))
    dstm = jnp.pad(dst.reshape(NW, NBLK, EB),
                   ((0, 0), (0, NBLKP - NBLK), (0, 0)))
    srcg = jnp.pad(src.reshape(NW, GR, LANES),
                   ((0, 0), (0, GRP - GR), (0, 0)))
    dstg = jnp.pad(dst.reshape(NW, GR, LANES),
                   ((0, 0), (0, GRP - GR), (0, 0)))

    bxm, bym, bzm = _geom_call(
        atom_pos[:, 0], atom_pos[:, 1], atom_pos[:, 2], srcg, dstg)
    bx = bxm[:, :GR].reshape(E)
    by = bym[:, :GR].reshape(E)
    bz = bzm[:, :GR].reshape(E)

    d2 = bx * bx + by * by + bz * bz + 1e-12
    d = jnp.sqrt(d2)
    u = d / CUTOFF
    env = jnp.where(
        u < 1.0, 1.0 - 21.0 * u**5 + 35.0 * u**6 - 15.0 * u**7, 0.0)
    nvec = jnp.arange(1, RBF_DIM + 1, dtype=jnp.float32)
    rbf = (jnp.sqrt(2.0 / CUTOFF)
           * jnp.sin(nvec[None, :] * jnp.pi * u[:, None]) / d[:, None])
    fE = rbf * env[:, None]
    ux, uy, uz = bx / d, by / d, bz / d
    s3 = jnp.sqrt(3.0)
    Y9 = jnp.stack([
        jnp.ones_like(ux), ux, uy, uz,
        s3 * ux * uy, s3 * uy * uz, 0.5 * (3.0 * uz * uz - 1.0),
        s3 * ux * uz, 0.5 * s3 * (ux * ux - uy * uy),
    ], axis=1)

    zeros_c = jnp.zeros((N, WC), jnp.float32)

    def padc(arr):
        w = arr.shape[1]
        if w == WC:
            return arr
        return jnp.pad(arr, ((0, 0), (0, WC - w)))

    AS, AK, AB = [], [], []
    for l in range(NLAYER):
        AS.append(fE @ Wf[l][:, :H] + env[:, None] * bf[l][None, :H])
        aks, abs_ = [], []
        for (j0, j1) in _VCHUNKS:
            cck = _cc[j0:j1]
            compk = _compv[j0:j1]
            wf2 = Wf[l][:, H + cck]
            bf2 = bf[l][H + cck]
            yk = Y9[:, compk]
            a_k = padc((fE @ wf2 + env[:, None] * bf2[None, :]) * yk)
            if l == 0:
                aks.append(a_k)
            else:
                wf3 = Wf[l][:, H + C + cck]
                bf3 = bf[l][H + C + cck]
                b_k = padc(fE @ wf3 + env[:, None] * bf3[None, :])
                abs_.append(_pack2(a_k, b_k))
        AK.append(aks)
        AB.append(abs_)

    s = x
    v = jnp.zeros((N, V), jnp.float32)
    for l in range(NLAYER):
        phi = _silu(s @ Wp1[l] + bp1[l]) @ Wp2[l] + bp2[l]

        out = _seg_call(False, phi[:, :H], AS[l], zeros_c, srcm, dstm)
        ds = out[0] + out[1]
        s = s + ds

        dv_parts = []
        for ci, (j0, j1) in enumerate(_VCHUNKS):
            w = j1 - j0
            cck = _cc[j0:j1]
            t1k = padc(phi[:, H + cck])
            if l == 0:
                out = _seg_call(False, t1k, AK[l][ci], zeros_c, srcm, dstm)
            else:
                t2k = padc(phi[:, H + C + cck] * v[:, j0:j1])
                out = _seg_call(True, _pack2(t1k, t2k), AB[l][ci],
                                zeros_c, srcm, dstm)
            dv_parts.append((out[0] + out[1])[:, :w])
        v = v + jnp.concatenate(dv_parts, axis=1)

        vn = jnp.concatenate([
            jnp.sqrt(v[:, :128] ** 2 + 1e-12),
            jnp.sqrt(jnp.sum(v[:, 128:320].reshape(N, 64, 3) ** 2, -1)
                     + 1e-12),
            jnp.sqrt(jnp.sum(v[:, 320:480].reshape(N, 32, 5) ** 2, -1)
                     + 1e-12),
        ], axis=1)
        a = (_silu(jnp.concatenate([s, vn], axis=1) @ Wu1[l] + bu1[l])
             @ Wu2[l] + bu2[l])
        s = s + a[:, :H]
        v = v + a[:, H:][:, cc] * v
    return s


# speedup vs baseline: 1.0389x; 1.0003x over previous
import functools

import jax
import jax.numpy as jnp
import numpy as np
from jax import lax
from jax.experimental import pallas as pl
from jax.experimental.pallas import tpu as pltpu
from jax.experimental.pallas import tpu_sc as plsc

N = 10000
E = 320000
H = 128
RBF_DIM = 20
NLAYER = 3
CUTOFF = 5.0
C = 224
V = 480
G = H + 2 * C

NC = 2
NS = 16
NW = NC * NS
LANES = 16

EB = 40
EPT = E // NW
NBLK = EPT // EB
NBLKP = 256

GR = EPT // LANES
GRP = 640

WC = 128
_VCHUNKS = [(0, 128), (128, 256), (256, 384), (384, 480)]

_cc = np.concatenate([
    np.arange(128),
    np.repeat(128 + np.arange(64), 3),
    np.repeat(192 + np.arange(32), 5),
]).astype(np.int32)
_compv = np.concatenate([
    np.zeros(128, np.int32),
    1 + np.tile(np.arange(3, dtype=np.int32), 64),
    4 + np.tile(np.arange(5, dtype=np.int32), 32),
])


def _mesh():
    return plsc.VectorSubcoreMesh(
        core_axis_name="c", subcore_axis_name="s",
        num_cores=NC, num_subcores=NS)


def _worker_id():
    return lax.axis_index("c") * NS + lax.axis_index("s")


def _geom_body(posx_hbm, posy_hbm, posz_hbm, srcg_hbm, dstg_hbm,
               bx_hbm, by_hbm, bz_hbm,
               tabx, taby, tabz, src8, dst8, ox8, oy8, oz8):
    wid = _worker_id()
    pltpu.sync_copy(posx_hbm, tabx)
    pltpu.sync_copy(posy_hbm, taby)
    pltpu.sync_copy(posz_hbm, tabz)

    def group(g, carry):
        pltpu.sync_copy(srcg_hbm.at[wid, pl.ds(g * 8, 8)], src8)
        pltpu.sync_copy(dstg_hbm.at[wid, pl.ds(g * 8, 8)], dst8)
        for k in range(8):
            isrc = src8[k]
            idst = dst8[k]
            ox8[k] = (plsc.load_gather(tabx, [isrc])
                      - plsc.load_gather(tabx, [idst]))
            oy8[k] = (plsc.load_gather(taby, [isrc])
                      - plsc.load_gather(taby, [idst]))
            oz8[k] = (plsc.load_gather(tabz, [isrc])
                      - plsc.load_gather(tabz, [idst]))
        pltpu.sync_copy(ox8, bx_hbm.at[wid, pl.ds(g * 8, 8)])
        pltpu.sync_copy(oy8, by_hbm.at[wid, pl.ds(g * 8, 8)])
        pltpu.sync_copy(oz8, bz_hbm.at[wid, pl.ds(g * 8, 8)])
        return carry

    lax.fori_loop(0, GRP // 8, group, 0)


@functools.lru_cache(maxsize=None)
def _build_geom_kernel():
    o3 = jax.ShapeDtypeStruct((NW, GRP, LANES), jnp.float32)
    return functools.partial(
        pl.kernel,
        out_type=(o3, o3, o3),
        mesh=_mesh(),
        compiler_params=pltpu.CompilerParams(needs_layout_passes=False),
        scratch_types=[
            pltpu.VMEM((N,), jnp.float32),
            pltpu.VMEM((N,), jnp.float32),
            pltpu.VMEM((N,), jnp.float32),
            pltpu.VMEM((8, LANES), jnp.int32),
            pltpu.VMEM((8, LANES), jnp.int32),
            pltpu.VMEM((8, LANES), jnp.float32),
            pltpu.VMEM((8, LANES), jnp.float32),
            pltpu.VMEM((8, LANES), jnp.float32),
        ],
    )(_geom_body)


def _geom_call(posx, posy, posz, srcg, dstg):
    return _build_geom_kernel()(posx, posy, posz, srcg, dstg)


def _make_seg_kernel(has_b):
    nj = WC // LANES
    eb = EB
    nblk = EPT // eb
    nblkp = ((nblk + 7) // 8) * 8

    def body(*refs):
        if has_b:
            (t1_hbm, a_hbm, zero_hbm, srcm_hbm, dstm_hbm,
             out_hbm, srcx, dstx, a_v, r1_v, msg_v, accum,
             sem0, sem1) = refs
        else:
            (t1_hbm, a_hbm, zero_hbm, srcm_hbm, dstm_hbm,
             out_hbm, srcx, dstx, a_v, r1_v, accum, sem0, sem1) = refs
            msg_v = None
        sems = (sem0, sem1)
        cid = lax.axis_index("c")
        sid = lax.axis_index("s")
        wid = cid * NS + sid

        @pl.when(sid == 0)
        def _init():
            pltpu.sync_copy(zero_hbm, accum)

        plsc.subcore_barrier()

        def issue(b, gslot, k, ph):
            base = wid * EPT + b * eb
            idx = srcx.at[gslot, k]
            pltpu.async_copy(t1_hbm.at[idx], r1_v.at[ph], sems[ph])
            pltpu.async_copy(a_hbm.at[pl.ds(base, eb)], a_v.at[ph],
                             sems[ph])

        def drain(b, gslot, k, ph):
            base = wid * EPT + b * eb
            idx = srcx.at[gslot, k]
            pltpu.make_async_copy(t1_hbm.at[idx], r1_v.at[ph],
                                  sems[ph]).wait()
            pltpu.make_async_copy(a_hbm.at[pl.ds(base, eb)], a_v.at[ph],
                                  sems[ph]).wait()

        hi_mask = np.int32(np.uint32(0xFFFF0000).view(np.int32))

        def compute(ph):
            def row(i, carry2):
                for j in range(nj):
                    sl = pl.ds(j * LANES, LANES)
                    if has_b:
                        ab = a_v[ph, i, sl]
                        t12 = r1_v[ph, i, sl]
                        af = plsc.bitcast(ab << 16, jnp.float32)
                        bf16_hi = plsc.bitcast(ab & hi_mask, jnp.float32)
                        t1f = plsc.bitcast(t12 << 16, jnp.float32)
                        t2f = plsc.bitcast(t12 & hi_mask, jnp.float32)
                        msg_v[i, sl] = af * t1f + bf16_hi * t2f
                    else:
                        r1_v[ph, i, sl] = a_v[ph, i, sl] * r1_v[ph, i, sl]
                return carry2

            lax.fori_loop(0, EB, row, 0)

        pltpu.sync_copy(srcm_hbm.at[wid, pl.ds(0, 8)], srcx.at[0])
        pltpu.sync_copy(dstm_hbm.at[wid, pl.ds(0, 8)], dstx.at[0])
        issue(0, 0, 0, 0)

        def group(g, carry):
            gcur = g & 1
            gnext = 1 - gcur

            @pl.when((g + 1) * 8 < nblkp)
            def _prefetch_idx():
                pltpu.sync_copy(srcm_hbm.at[wid, pl.ds((g + 1) * 8, 8)],
                                srcx.at[gnext])
                pltpu.sync_copy(dstm_hbm.at[wid, pl.ds((g + 1) * 8, 8)],
                                dstx.at[gnext])
            for k in range(8):
                b = g * 8 + k
                ph = k & 1

                @pl.when(b + 1 < nblk)
                def _issue_next(b=b, k=k, ph=ph, gcur=gcur, gnext=gnext):
                    if k + 1 < 8:
                        issue(b + 1, gcur, k + 1, 1 - ph)
                    else:
                        issue(b + 1, gnext, 0, 1 - ph)

                @pl.when(b < nblk)
                def _process(b=b, k=k, ph=ph, gcur=gcur):
                    drain(b, gcur, k, ph)
                    compute(ph)
                    msg = msg_v if has_b else r1_v.at[ph]
                    pltpu.sync_copy(msg,
                                    accum.at[dstx.at[gcur, k]], add=True)
            return carry

        lax.fori_loop(0, nblkp // 8, group, 0)
        plsc.subcore_barrier()

        @pl.when(sid == 0)
        def _writeout():
            pltpu.sync_copy(accum, out_hbm.at[cid])

    dt = jnp.int32 if has_b else jnp.float32
    scratch = [
        pltpu.VMEM((2, 8, eb), jnp.int32),
        pltpu.VMEM((2, 8, eb), jnp.int32),
        pltpu.VMEM((2, eb, WC), dt),
        pltpu.VMEM((2, eb, WC), dt),
    ]
    if has_b:
        scratch.append(pltpu.VMEM((eb, WC), jnp.float32))
    scratch += [
        pltpu.VMEM_SHARED((N, WC), jnp.float32),
        pltpu.SemaphoreType.DMA,
        pltpu.SemaphoreType.DMA,
    ]
    return functools.partial(
        pl.kernel,
        out_type=jax.ShapeDtypeStruct((NC, N, WC), jnp.float32),
        mesh=_mesh(),
        compiler_params=pltpu.CompilerParams(needs_layout_passes=False),
        scratch_types=scratch,
    )(body)


@functools.lru_cache(maxsize=None)
def _seg_kernel_cached(has_b):
    return _make_seg_kernel(has_b)


def _seg_call(has_b, *args):
    return _seg_kernel_cached(has_b)(*args)


def _silu(x):
    return x * jax.nn.sigmoid(x)


def _pack2(a, b):
    au = jax.lax.bitcast_convert_type(
        a.astype(jnp.bfloat16), jnp.uint16).astype(jnp.uint32)
    bu = jax.lax.bitcast_convert_type(
        b.astype(jnp.bfloat16), jnp.uint16).astype(jnp.uint32)
    return jax.lax.bitcast_convert_type(au | (bu << 16), jnp.int32)


def kernel(atom_pos, x, Wp1, bp1, Wp2, bp2, Wf, bf, Wu1, bu1, Wu2, bu2,
           edge_index):
    cc = jnp.asarray(_cc)

    src = edge_index[0].astype(jnp.int32)
    dst = edge_index[1].astype(jnp.int32)
    srcm = jnp.pad(src.reshape(NW, NBLK, EB),
                   ((0, 0), (0, NBLKP - NBLK), (0, 0)))
    dstm = jnp.pad(dst.reshape(NW, NBLK, EB),
                   ((0, 0), (0, NBLKP - NBLK), (0, 0)))
    srcg = jnp.pad(src.reshape(NW, GR, LANES),
                   ((0, 0), (0, GRP - GR), (0, 0)))
    dstg = jnp.pad(dst.reshape(NW, GR, LANES),
                   ((0, 0), (0, GRP - GR), (0, 0)))

    bxm, bym, bzm = _geom_call(
        atom_pos[:, 0], atom_pos[:, 1], atom_pos[:, 2], srcg, dstg)
    bx = bxm[:, :GR].reshape(E)
    by = bym[:, :GR].reshape(E)
    bz = bzm[:, :GR].reshape(E)

    d2 = bx * bx + by * by + bz * bz + 1e-12
    d = jnp.sqrt(d2)
    u = d / CUTOFF
    env = jnp.where(
        u < 1.0, 1.0 - 21.0 * u**5 + 35.0 * u**6 - 15.0 * u**7, 0.0)
    nvec = jnp.arange(1, RBF_DIM + 1, dtype=jnp.float32)
    rbf = (jnp.sqrt(2.0 / CUTOFF)
           * jnp.sin(nvec[None, :] * jnp.pi * u[:, None]) / d[:, None])
    fE = rbf * env[:, None]
    ux, uy, uz = bx / d, by / d, bz / d
    s3 = jnp.sqrt(3.0)
    Y9 = jnp.stack([
        jnp.ones_like(ux), ux, uy, uz,
        s3 * ux * uy, s3 * uy * uz, 0.5 * (3.0 * uz * uz - 1.0),
        s3 * ux * uz, 0.5 * s3 * (ux * ux - uy * uy),
    ], axis=1)

    zeros_c = jnp.zeros((N, WC), jnp.float32)

    def padc(arr):
        w = arr.shape[1]
        if w == WC:
            return arr
        return jnp.pad(arr, ((0, 0), (0, WC - w)))

    AS, AK, AB = [], [], []
    for l in range(NLAYER):
        AS.append(fE @ Wf[l][:, :H] + env[:, None] * bf[l][None, :H])
        aks, abs_ = [], []
        for (j0, j1) in _VCHUNKS:
            cck = _cc[j0:j1]
            compk = _compv[j0:j1]
            wf2 = Wf[l][:, H + cck]
            bf2 = bf[l][H + cck]
            yk = Y9[:, compk]
            a_k = padc((fE @ wf2 + env[:, None] * bf2[None, :]) * yk)
            if l == 0:
                aks.append(a_k)
            else:
                wf3 = Wf[l][:, H + C + cck]
                bf3 = bf[l][H + C + cck]
                b_k = padc(fE @ wf3 + env[:, None] * bf3[None, :])
                abs_.append(_pack2(a_k, b_k))
        AK.append(aks)
        AB.append(abs_)

    s = x
    v = jnp.zeros((N, V), jnp.float32)
    for l in range(NLAYER):
        phi = _silu(s @ Wp1[l] + bp1[l]) @ Wp2[l] + bp2[l]

        out = _seg_call(False, phi[:, :H], AS[l], zeros_c, srcm, dstm)
        ds = out[0] + out[1]
        s = s + ds

        dv_parts = []
        for ci, (j0, j1) in enumerate(_VCHUNKS):
            w = j1 - j0
            cck = _cc[j0:j1]
            t1k = padc(phi[:, H + cck])
            if l == 0:
                out = _seg_call(False, t1k, AK[l][ci], zeros_c, srcm, dstm)
            else:
                t2k = padc(phi[:, H + C + cck] * v[:, j0:j1])
                out = _seg_call(True, _pack2(t1k, t2k), AB[l][ci],
                                zeros_c, srcm, dstm)
            dv_parts.append((out[0] + out[1])[:, :w])
        v = v + jnp.concatenate(dv_parts, axis=1)

        vn = jnp.concatenate([
            jnp.sqrt(v[:, :128] ** 2 + 1e-12),
            jnp.sqrt(jnp.sum(v[:, 128:320].reshape(N, 64, 3) ** 2, -1)
                     + 1e-12),
            jnp.sqrt(jnp.sum(v[:, 320:480].reshape(N, 32, 5) ** 2, -1)
                     + 1e-12),
        ], axis=1)
        a = (_silu(jnp.concatenate([s, vn], axis=1) @ Wu1[l] + bu1[l])
             @ Wu2[l] + bu2[l])
        s = s + a[:, :H]
        v = v + a[:, H:][:, cc] * v
    return s
